# Initial kernel scaffold; baseline (speedup 1.0000x reference)
import jax
import jax.numpy as jnp
from jax.experimental import pallas as pl


def kernel(x, router_w, router_b, w1, b1, w2, b2):
    raise NotImplementedError("write your pallas kernel here")


# speedup vs baseline: 2.1114x; 2.1114x over previous
import functools

import jax
import jax.numpy as jnp
from jax import lax
from jax.experimental import pallas as pl
from jax.experimental.pallas import tpu as pltpu
from jax.experimental.pallas import tpu_sc as plsc

INTERPRET = False
USE_SC = True
BF16_ROUTER = True

E = 8
K = 2
D = 1024
F = 2048
T = 4096
EP = 128
BS = 256
G = T * K // BS + (E - 1)
P = G * BS
GP = 48
NW = 32
APW = T * K // NW
TPW = T // NW
DW = D // 2

NEG = -1e30


def _gelu(h):
    return 0.5 * h * (1.0 + jax.lax.erf(h * 0.7071067811865476))


def _router_body(x_ref, rw_ref, rb_ref, probs_ref, widx_ref, xbf_ref):
    xb = x_ref[...]
    rw = rw_ref[...]
    if BF16_ROUTER:
        lg = jnp.dot(xb.astype(jnp.bfloat16), rw.astype(jnp.bfloat16),
                     preferred_element_type=jnp.float32)
    else:
        lg = jnp.dot(xb, rw, preferred_element_type=jnp.float32,
                     precision=jax.lax.Precision.HIGHEST)
    lg = lg + rb_ref[...]
    col = jax.lax.broadcasted_iota(jnp.int32, lg.shape, 1)
    lg = jnp.where(col < E, lg, NEG)
    v0 = jnp.max(lg, axis=1, keepdims=True)
    i0 = jnp.min(jnp.where(lg == v0, col, EP), axis=1, keepdims=True)
    lg1 = jnp.where(col == i0, NEG, lg)
    v1 = jnp.max(lg1, axis=1, keepdims=True)
    i1 = jnp.min(jnp.where(lg1 == v1, col, EP), axis=1, keepdims=True)
    p0 = 1.0 / (1.0 + jnp.exp(v1 - v0))
    p1 = 1.0 - p0
    probs_ref[...] = jnp.where(col == 0, p0, jnp.where(col == 1, p1, 0.0))
    widx_ref[...] = jnp.where(col == 0, i0, jnp.where(col == 1, i1, 0))
    xbf_ref[...] = xb.astype(jnp.bfloat16)


def _router(x2d, rw_pad, rb_pad):
    TB = 1024
    return pl.pallas_call(
        _router_body,
        grid=(T // TB,),
        in_specs=[
            pl.BlockSpec((TB, D), lambda i: (i, 0)),
            pl.BlockSpec((D, EP), lambda i: (0, 0)),
            pl.BlockSpec((1, EP), lambda i: (0, 0)),
        ],
        out_specs=[
            pl.BlockSpec((TB, EP), lambda i: (i, 0)),
            pl.BlockSpec((TB, EP), lambda i: (i, 0)),
            pl.BlockSpec((TB, D), lambda i: (i, 0)),
        ],
        out_shape=[
            jax.ShapeDtypeStruct((T, EP), jnp.float32),
            jax.ShapeDtypeStruct((T, EP), jnp.int32),
            jax.ShapeDtypeStruct((T, D), jnp.bfloat16),
        ],
        interpret=INTERPRET,
    )(x2d, rw_pad, rb_pad)


def _lane_iota():
    return lax.iota(jnp.int32, 16)


def _extract(vec, lane):
    return jnp.sum(jnp.where(_lane_iota() == lane, vec, 0))


def _binning_kernel(easgn_hbm, x_hbm, xs_hbm, dest_hbm, blkexp_hbm,
                    easgn_v, dest_v, tok_v, dst_v, rows_a, rows_b, blk_v,
                    sem_g, sem_s):
    wid = lax.axis_index("s") * 2 + lax.axis_index("c")
    lane = _lane_iota()

    pltpu.sync_copy(easgn_hbm, easgn_v)

    my_start = wid * (APW // 16)

    def scan_body(c, carry):
        acc, snap = carry
        snap = [jnp.where(c == my_start, a, s) for a, s in zip(acc, snap)]
        v = easgn_v[pl.ds(c * 16, 16)]
        acc = [a + jnp.where(v == e, 1, 0) for e, a in enumerate(acc)]
        return acc, snap

    zeros16 = jnp.zeros((16,), jnp.int32)
    acc, snap = lax.fori_loop(0, (T * K) // 16, scan_body,
                              ([zeros16] * E, [zeros16] * E))
    counts = jnp.zeros((16,), jnp.int32)
    prefix = jnp.zeros((16,), jnp.int32)
    for e in range(E):
        counts = counts + jnp.where(lane == e, jnp.sum(acc[e]), 0)
        prefix = prefix + jnp.where(lane == e, jnp.sum(snap[e]), 0)

    nblk = (counts + (BS - 1)) // BS
    cumblk = plsc.cumsum(nblk)
    excl = (cumblk - nblk) * BS
    base_vec = excl + prefix

    @pl.when(wid == 0)
    def _():
        cb = [_extract(cumblk, e) for e in range(E)]
        for cc in range(GP // 16):
            gvec = lane + 16 * cc
            val = jnp.zeros((16,), jnp.int32)
            for e in range(E):
                val = val + jnp.where(cb[e] <= gvec, 1, 0)
            blk_v[pl.ds(cc * 16, 16)] = jnp.minimum(val, E - 1)
        pltpu.sync_copy(blk_v, blkexp_hbm)

    base = [_extract(base_vec, e) for e in range(E)]
    running = [jnp.int32(0)] * E
    for c in range(APW // 16):
        v = easgn_v[pl.ds((my_start + c) * 16, 16)]
        destc = jnp.zeros((16,), jnp.int32)
        for e in range(E):
            m = v == e
            ones = jnp.where(m, 1, 0)
            r = plsc.cumsum(ones)
            pos = (base[e] + running[e] - 1) + r
            destc = jnp.where(m, pos, destc)
            running[e] = running[e] + jnp.max(r)
        dest_v[pl.ds(c * 16, 16)] = destc
    pltpu.sync_copy(dest_v, dest_hbm.at[pl.ds(wid * APW, APW)])

    NB = 64
    gbase = wid * APW

    def fill_idx(b):
        for c in range(NB // 16):
            gi = gbase + b * NB + c * 16 + lane
            tok = jnp.bitwise_and(gi, T - 1)
            tok_v[pl.ds(c * 16, 16)] = tok
            dst_v[pl.ds(c * 16, 16)] = dest_v[pl.ds(b * NB + c * 16, 16)]

    bufs = [rows_a, rows_b]
    pend_scatter = [None, None]
    pend_gather = [None, None]
    for b in range(APW // NB):
        sl = b % 2
        if pend_scatter[sl] is not None:
            pend_scatter[sl].wait()
        fill_idx(b)
        cp = pltpu.async_copy(x_hbm.at[tok_v], bufs[sl], sem_g)
        cp.wait()
        sc = pltpu.async_copy(bufs[sl], xs_hbm.at[dst_v], sem_s)
        sc.wait()


def _run_binning(easgn, xi32):
    mesh = plsc.VectorSubcoreMesh(core_axis_name="c", subcore_axis_name="s")
    kern = pl.kernel(
        _binning_kernel,
        mesh=mesh,
        compiler_params=pltpu.CompilerParams(needs_layout_passes=False),
        out_type=[
            jax.ShapeDtypeStruct((P, DW), jnp.int32),
            jax.ShapeDtypeStruct((T * K,), jnp.int32),
            jax.ShapeDtypeStruct((GP,), jnp.int32),
        ],
        scratch_types=[
            pltpu.VMEM((T * K,), jnp.int32),
            pltpu.VMEM((APW,), jnp.int32),
            pltpu.VMEM((64,), jnp.int32),
            pltpu.VMEM((64,), jnp.int32),
            pltpu.VMEM((64, DW), jnp.int32),
            pltpu.VMEM((64, DW), jnp.int32),
            pltpu.VMEM((GP,), jnp.int32),
            pltpu.SemaphoreType.DMA,
            pltpu.SemaphoreType.DMA,
        ],
    )
    return kern(easgn, xi32)


def _binning_jnp(easgn, xi32):
    a = easgn
    onehot = (a[:, None] == jnp.arange(E)[None, :]).astype(jnp.int32)
    counts = jnp.sum(onehot, axis=0)
    nblk = (counts + BS - 1) // BS
    cumblk = jnp.cumsum(nblk)
    excl = (cumblk - nblk) * BS
    rank = jnp.cumsum(onehot, axis=0) - onehot
    dest = excl[a] + jnp.take_along_axis(rank, a[:, None], axis=1)[:, 0]
    xs = jnp.zeros((P, DW), jnp.int32).at[dest].set(
        xi32[jnp.arange(T * K) % T])
    blkexp = jnp.minimum(
        jnp.sum(cumblk[None, :] <= jnp.arange(GP)[:, None], axis=1), E - 1
    ).astype(jnp.int32)
    return xs, dest, blkexp


def _mlp_body(be_ref, xs_ref, w1_ref, b1_ref, w2_ref, b2_ref, ys_ref):
    h = jnp.dot(xs_ref[...], w1_ref[0], preferred_element_type=jnp.float32)
    h = _gelu(h + b1_ref[0])
    y = jnp.dot(h.astype(jnp.bfloat16), w2_ref[0],
                preferred_element_type=jnp.float32)
    ys_ref[...] = y + b2_ref[0]


def _grouped_mlp(blkexp, xs_bf, w1b, b1, w2b, b2):
    grid_spec = pltpu.PrefetchScalarGridSpec(
        num_scalar_prefetch=1,
        grid=(G,),
        in_specs=[
            pl.BlockSpec((BS, D), lambda g, be: (g, 0)),
            pl.BlockSpec((1, D, F), lambda g, be: (be[g], 0, 0)),
            pl.BlockSpec((1, 1, F), lambda g, be: (be[g], 0, 0)),
            pl.BlockSpec((1, F, D), lambda g, be: (be[g], 0, 0)),
            pl.BlockSpec((1, 1, D), lambda g, be: (be[g], 0, 0)),
        ],
        out_specs=pl.BlockSpec((BS, D), lambda g, be: (g, 0)),
    )
    return pl.pallas_call(
        _mlp_body,
        grid_spec=grid_spec,
        out_shape=jax.ShapeDtypeStruct((P, D), jnp.float32),
        interpret=INTERPRET,
    )(blkexp, xs_bf, w1b, b1, w2b, b2)


def _combine_kernel(ys_hbm, dest_hbm, pw_hbm, out_hbm,
                    d0_v, d1_v, p0_v, p1_v, r0_v, r1_v, o_v, sem0, sem1):
    wid = lax.axis_index("s") * 2 + lax.axis_index("c")
    tb = wid * TPW
    pltpu.sync_copy(dest_hbm.at[pl.ds(tb, TPW)], d0_v)
    pltpu.sync_copy(dest_hbm.at[pl.ds(T + tb, TPW)], d1_v)
    pltpu.sync_copy(pw_hbm.at[pl.ds(tb, TPW)], p0_v)
    pltpu.sync_copy(pw_hbm.at[pl.ds(T + tb, TPW)], p1_v)
    lane = _lane_iota()
    for c in range(TPW // 16):
        i0 = d0_v[pl.ds(c * 16, 16)]
        i1 = d1_v[pl.ds(c * 16, 16)]
        cp0 = pltpu.async_copy(ys_hbm.at[i0], r0_v, sem0)
        cp1 = pltpu.async_copy(ys_hbm.at[i1], r1_v, sem1)
        cp0.wait()
        cp1.wait()
        pa = p0_v[pl.ds(c * 16, 16)]
        pb = p1_v[pl.ds(c * 16, 16)]
        for i in range(16):
            s0 = lax.reduce_sum_p.bind(
                jnp.where(lane == i, pa, 0.0), axes=(0,))
            s1 = lax.reduce_sum_p.bind(
                jnp.where(lane == i, pb, 0.0), axes=(0,))

            def col_body(j, _):
                for u in range(4):
                    o_v[i, pl.ds(j * 64 + u * 16, 16)] = (
                        r0_v[i, pl.ds(j * 64 + u * 16, 16)] * s0
                        + r1_v[i, pl.ds(j * 64 + u * 16, 16)] * s1)
                return 0

            lax.fori_loop(0, D // 64, col_body, 0)
        pltpu.sync_copy(o_v, out_hbm.at[pl.ds(tb + c * 16, 16)])


def _run_combine(ys, dest, pw):
    mesh = plsc.VectorSubcoreMesh(core_axis_name="c", subcore_axis_name="s")
    kern = pl.kernel(
        _combine_kernel,
        mesh=mesh,
        compiler_params=pltpu.CompilerParams(needs_layout_passes=False),
        out_type=jax.ShapeDtypeStruct((T, D), jnp.float32),
        scratch_types=[
            pltpu.VMEM((TPW,), jnp.int32),
            pltpu.VMEM((TPW,), jnp.int32),
            pltpu.VMEM((TPW,), jnp.float32),
            pltpu.VMEM((TPW,), jnp.float32),
            pltpu.VMEM((16, D), jnp.float32),
            pltpu.VMEM((16, D), jnp.float32),
            pltpu.VMEM((16, D), jnp.float32),
            pltpu.SemaphoreType.DMA,
            pltpu.SemaphoreType.DMA,
        ],
    )
    return kern(ys, dest, pw)


def _combine_jnp(ys, dest, pw):
    r0 = ys[dest[:T]]
    r1 = ys[dest[T:]]
    return r0 * pw[:T, None] + r1 * pw[T:, None]


def kernel(x, router_w, router_b, w1, b1, w2, b2):
    B, S, _ = x.shape
    x2d = x.reshape(T, D)
    rw_pad = jnp.pad(router_w, ((0, 0), (0, EP - E)))
    rb_pad = jnp.pad(router_b, (0, EP - E)).reshape(1, EP)

    probs_pad, widx, x_bf = _router(x2d, rw_pad, rb_pad)

    easgn = jnp.concatenate([widx[:, 0], widx[:, 1]])
    pw = jnp.concatenate([probs_pad[:, 0], probs_pad[:, 1]])
    xi32 = lax.bitcast_convert_type(
        x_bf.reshape(T, DW, 2), jnp.int32)

    if USE_SC:
        xs_i32, dest, blkexp = _run_binning(easgn, xi32)
    else:
        xs_i32, dest, blkexp = _binning_jnp(easgn, xi32)

    xs_bf = lax.bitcast_convert_type(xs_i32, jnp.bfloat16).reshape(P, D)
    w1b = w1.astype(jnp.bfloat16)
    w2b = w2.astype(jnp.bfloat16)
    ys = _grouped_mlp(blkexp[:G], xs_bf, w1b, b1.reshape(E, 1, F),
                      w2b, b2.reshape(E, 1, D))

    if USE_SC:
        out2d = _run_combine(ys, dest, pw)
    else:
        out2d = _combine_jnp(ys, dest, pw)

    out = out2d.reshape(B, S, D)
    probs = probs_pad[:, :K].reshape(B, S, K)
    return out, probs


# speedup vs baseline: 4.1788x; 1.9791x over previous
import jax
import jax.numpy as jnp
from jax import lax
from jax.experimental import pallas as pl
from jax.experimental.pallas import tpu as pltpu
from jax.experimental.pallas import tpu_sc as plsc

INTERPRET = False
USE_SC = True

E = 8
K = 2
D = 1024
F = 2048
T = 4096
EP = 128
BS = 256
G = T * K // BS + (E - 1)
P = G * BS
GP = 48
NW = 32
APW = T * K // NW
TPW = T // NW

NEG = -1e30


def _gelu(h):
    return 0.5 * h * (1.0 + jax.lax.erf(h * 0.7071067811865476))


def _router_body(x_ref, rw_ref, rb_ref, eidx_ref, pval_ref):
    xb = x_ref[...]
    rw = rw_ref[...]
    lg = lax.dot_general(
        rw.astype(jnp.bfloat16), xb.astype(jnp.bfloat16),
        (((0,), (1,)), ((), ())),
        preferred_element_type=jnp.float32)
    lg = lg + rb_ref[...]
    row = jax.lax.broadcasted_iota(jnp.int32, lg.shape, 0)
    lg = jnp.where(row < E, lg, NEG)
    v0 = jnp.max(lg, axis=0, keepdims=True)
    i0 = jnp.min(jnp.where(lg == v0, row, EP), axis=0, keepdims=True)
    lg1 = jnp.where(row == i0, NEG, lg)
    v1 = jnp.max(lg1, axis=0, keepdims=True)
    i1 = jnp.min(jnp.where(lg1 == v1, row, EP), axis=0, keepdims=True)
    p0 = 1.0 / (1.0 + jnp.exp(v1 - v0))
    p1 = 1.0 - p0
    zi = jnp.zeros_like(i0)
    zp = jnp.zeros_like(p0)
    eidx_ref[...] = jnp.concatenate([i0, i1, zi, zi, zi, zi, zi, zi], axis=0)
    pval_ref[...] = jnp.concatenate([p0, p1, zp, zp, zp, zp, zp, zp], axis=0)


def _router(x2d, rw_pad, rb_t):
    TB = 1024
    return pl.pallas_call(
        _router_body,
        grid=(T // TB,),
        in_specs=[
            pl.BlockSpec((TB, D), lambda i: (i, 0)),
            pl.BlockSpec((D, EP), lambda i: (0, 0)),
            pl.BlockSpec((EP, 1), lambda i: (0, 0)),
        ],
        out_specs=[
            pl.BlockSpec((8, TB), lambda i: (0, i)),
            pl.BlockSpec((8, TB), lambda i: (0, i)),
        ],
        out_shape=[
            jax.ShapeDtypeStruct((8, T), jnp.int32),
            jax.ShapeDtypeStruct((8, T), jnp.float32),
        ],
        interpret=INTERPRET,
    )(x2d, rw_pad, rb_t)


def _lane_iota():
    return lax.iota(jnp.int32, 16)


def _extract(vec, lane):
    return jnp.sum(jnp.where(_lane_iota() == lane, vec, 0))


def _extract_f(vec, lane):
    return jnp.sum(jnp.where(_lane_iota() == lane, vec, 0.0))


def _binning_kernel(easgn_hbm, x_hbm, xs_hbm, dest_hbm, blkexp_hbm,
                    easgn_v, dest_v, tok_a, tok_b, dst_a, dst_b,
                    rows_a, rows_b, blk_v, sem_ga, sem_gb, sem_sa, sem_sb):
    wid = lax.axis_index("s") * 2 + lax.axis_index("c")
    lane = _lane_iota()

    pltpu.sync_copy(easgn_hbm, easgn_v)

    my_start = wid * (APW // 16)

    def scan_body(c, carry):
        acc, snap = carry
        snap = [jnp.where(c == my_start, a, s) for a, s in zip(acc, snap)]
        v = easgn_v[pl.ds(c * 16, 16)]
        acc = [a + jnp.where(v == e, 1, 0) for e, a in enumerate(acc)]
        return acc, snap

    zeros16 = jnp.zeros((16,), jnp.int32)
    acc, snap = lax.fori_loop(0, (T * K) // 16, scan_body,
                              ([zeros16] * E, [zeros16] * E))
    counts = jnp.zeros((16,), jnp.int32)
    prefix = jnp.zeros((16,), jnp.int32)
    for e in range(E):
        counts = counts + jnp.where(lane == e, jnp.sum(acc[e]), 0)
        prefix = prefix + jnp.where(lane == e, jnp.sum(snap[e]), 0)

    nblk = (counts + (BS - 1)) // BS
    cumblk = plsc.cumsum(nblk)
    excl = (cumblk - nblk) * BS
    base_vec = excl + prefix

    @pl.when(wid == 0)
    def _():
        cb = [_extract(cumblk, e) for e in range(E)]
        for cc in range(GP // 16):
            gvec = lane + 16 * cc
            val = jnp.zeros((16,), jnp.int32)
            for e in range(E):
                val = val + jnp.where(cb[e] <= gvec, 1, 0)
            blk_v[pl.ds(cc * 16, 16)] = jnp.minimum(val, E - 1)
        pltpu.sync_copy(blk_v, blkexp_hbm)

    base = [_extract(base_vec, e) for e in range(E)]
    running = [jnp.int32(0)] * E
    for c in range(APW // 16):
        v = easgn_v[pl.ds((my_start + c) * 16, 16)]
        destc = jnp.zeros((16,), jnp.int32)
        for e in range(E):
            m = v == e
            ones = jnp.where(m, 1, 0)
            r = plsc.cumsum(ones)
            pos = (base[e] + running[e] - 1) + r
            destc = jnp.where(m, pos, destc)
            running[e] = running[e] + jnp.max(r)
        dest_v[pl.ds(c * 16, 16)] = destc
    pltpu.sync_copy(dest_v, dest_hbm.at[pl.ds(wid * APW, APW)])

    NB = 32
    NBATCH = APW // NB
    gbase = wid * APW
    toks = [tok_a, tok_b]
    dsts = [dst_a, dst_b]
    bufs = [rows_a, rows_b]
    gsems = [sem_ga, sem_gb]
    ssems = [sem_sa, sem_sb]

    def fill_idx(b, sl):
        for c in range(NB // 16):
            gi = gbase + b * NB + c * 16 + lane
            toks[sl][pl.ds(c * 16, 16)] = jnp.bitwise_and(gi, T - 1)
            dsts[sl][pl.ds(c * 16, 16)] = dest_v[pl.ds(b * NB + c * 16, 16)]

    pend_s = [None, None]
    for b in range(NBATCH):
        sl = b % 2
        if pend_s[sl] is not None:
            pend_s[sl].wait()
        fill_idx(b, sl)
        g = pltpu.async_copy(x_hbm.at[toks[sl]], bufs[sl], gsems[sl])
        g.wait()
        pend_s[sl] = pltpu.async_copy(bufs[sl], xs_hbm.at[dsts[sl]], ssems[sl])
    for sl in (0, 1):
        if pend_s[sl] is not None:
            pend_s[sl].wait()


def _run_binning(easgn, x2d):
    mesh = plsc.VectorSubcoreMesh(core_axis_name="c", subcore_axis_name="s")
    kern = pl.kernel(
        _binning_kernel,
        mesh=mesh,
        compiler_params=pltpu.CompilerParams(needs_layout_passes=False),
        out_type=[
            jax.ShapeDtypeStruct((P, D), jnp.float32),
            jax.ShapeDtypeStruct((T * K,), jnp.int32),
            jax.ShapeDtypeStruct((GP,), jnp.int32),
        ],
        scratch_types=[
            pltpu.VMEM((T * K,), jnp.int32),
            pltpu.VMEM((APW,), jnp.int32),
            pltpu.VMEM((32,), jnp.int32),
            pltpu.VMEM((32,), jnp.int32),
            pltpu.VMEM((32,), jnp.int32),
            pltpu.VMEM((32,), jnp.int32),
            pltpu.VMEM((32, D), jnp.float32),
            pltpu.VMEM((32, D), jnp.float32),
            pltpu.VMEM((GP,), jnp.int32),
            pltpu.SemaphoreType.DMA,
            pltpu.SemaphoreType.DMA,
            pltpu.SemaphoreType.DMA,
            pltpu.SemaphoreType.DMA,
        ],
    )
    return kern(easgn, x2d)


def _binning_jnp(easgn, x2d):
    a = easgn
    onehot = (a[:, None] == jnp.arange(E)[None, :]).astype(jnp.int32)
    counts = jnp.sum(onehot, axis=0)
    nblk = (counts + BS - 1) // BS
    cumblk = jnp.cumsum(nblk)
    excl = (cumblk - nblk) * BS
    rank = jnp.cumsum(onehot, axis=0) - onehot
    dest = excl[a] + jnp.take_along_axis(rank, a[:, None], axis=1)[:, 0]
    xs = jnp.zeros((P, D), jnp.float32).at[dest].set(
        x2d[jnp.arange(T * K) % T])
    blkexp = jnp.minimum(
        jnp.sum(cumblk[None, :] <= jnp.arange(GP)[:, None], axis=1), E - 1
    ).astype(jnp.int32)
    return xs, dest, blkexp


def _mlp_body(be_ref, xs_ref, w1_ref, b1_ref, w2_ref, b2_ref, ys_ref):
    h = jnp.dot(xs_ref[...].astype(jnp.bfloat16), w1_ref[0],
                preferred_element_type=jnp.float32)
    h = _gelu(h + b1_ref[0])
    y = jnp.dot(h.astype(jnp.bfloat16), w2_ref[0],
                preferred_element_type=jnp.float32)
    ys_ref[...] = y + b2_ref[0]


def _grouped_mlp(blkexp, xs, w1b, b1, w2b, b2):
    grid_spec = pltpu.PrefetchScalarGridSpec(
        num_scalar_prefetch=1,
        grid=(G,),
        in_specs=[
            pl.BlockSpec((BS, D), lambda g, be: (g, 0)),
            pl.BlockSpec((1, D, F), lambda g, be: (be[g], 0, 0)),
            pl.BlockSpec((1, 1, F), lambda g, be: (be[g], 0, 0)),
            pl.BlockSpec((1, F, D), lambda g, be: (be[g], 0, 0)),
            pl.BlockSpec((1, 1, D), lambda g, be: (be[g], 0, 0)),
        ],
        out_specs=pl.BlockSpec((BS, D), lambda g, be: (g, 0)),
    )
    return pl.pallas_call(
        _mlp_body,
        grid_spec=grid_spec,
        out_shape=jax.ShapeDtypeStruct((P, D), jnp.float32),
        interpret=INTERPRET,
    )(blkexp, xs, w1b, b1, w2b, b2)


def _combine_kernel(ys_hbm, dest_hbm, pw_hbm, out_hbm,
                    d0_v, d1_v, p0_v, p1_v, r0_v, r1_v, o_v, sem0, sem1):
    wid = lax.axis_index("s") * 2 + lax.axis_index("c")
    tb = wid * TPW
    pltpu.sync_copy(dest_hbm.at[pl.ds(tb, TPW)], d0_v)
    pltpu.sync_copy(dest_hbm.at[pl.ds(T + tb, TPW)], d1_v)
    pltpu.sync_copy(pw_hbm.at[pl.ds(tb, TPW)], p0_v)
    pltpu.sync_copy(pw_hbm.at[pl.ds(T + tb, TPW)], p1_v)
    for c in range(TPW // 16):
        i0 = d0_v[pl.ds(c * 16, 16)]
        i1 = d1_v[pl.ds(c * 16, 16)]
        cp0 = pltpu.async_copy(ys_hbm.at[i0], r0_v, sem0)
        cp1 = pltpu.async_copy(ys_hbm.at[i1], r1_v, sem1)
        cp0.wait()
        cp1.wait()
        pa = p0_v[pl.ds(c * 16, 16)]
        pb = p1_v[pl.ds(c * 16, 16)]
        for i in range(16):
            s0 = _extract_f(pa, i)
            s1 = _extract_f(pb, i)

            def col_body(j, _):
                for u in range(4):
                    o_v[i, pl.ds(j * 64 + u * 16, 16)] = (
                        r0_v[i, pl.ds(j * 64 + u * 16, 16)] * s0
                        + r1_v[i, pl.ds(j * 64 + u * 16, 16)] * s1)
                return 0

            lax.fori_loop(0, D // 64, col_body, 0)
        pltpu.sync_copy(o_v, out_hbm.at[pl.ds(tb + c * 16, 16)])


def _run_combine(ys, dest, pw):
    mesh = plsc.VectorSubcoreMesh(core_axis_name="c", subcore_axis_name="s")
    kern = pl.kernel(
        _combine_kernel,
        mesh=mesh,
        compiler_params=pltpu.CompilerParams(needs_layout_passes=False),
        out_type=jax.ShapeDtypeStruct((T, D), jnp.float32),
        scratch_types=[
            pltpu.VMEM((TPW,), jnp.int32),
            pltpu.VMEM((TPW,), jnp.int32),
            pltpu.VMEM((TPW,), jnp.float32),
            pltpu.VMEM((TPW,), jnp.float32),
            pltpu.VMEM((16, D), jnp.float32),
            pltpu.VMEM((16, D), jnp.float32),
            pltpu.VMEM((16, D), jnp.float32),
            pltpu.SemaphoreType.DMA,
            pltpu.SemaphoreType.DMA,
        ],
    )
    return kern(ys, dest, pw)


def _combine_jnp(ys, dest, pw):
    r0 = ys[dest[:T]]
    r1 = ys[dest[T:]]
    return r0 * pw[:T, None] + r1 * pw[T:, None]


def kernel(x, router_w, router_b, w1, b1, w2, b2):
    B, S, _ = x.shape
    x2d = x.reshape(T, D)
    rw_pad = jnp.pad(router_w, ((0, 0), (0, EP - E)))
    rb_t = jnp.pad(router_b, (0, EP - E)).reshape(EP, 1)

    eidx, pval = _router(x2d, rw_pad, rb_t)

    easgn = eidx[:2].reshape(T * K)
    pw = pval[:2].reshape(T * K)

    if USE_SC:
        xs, dest, blkexp = _run_binning(easgn, x2d)
    else:
        xs, dest, blkexp = _binning_jnp(easgn, x2d)

    w1b = w1.astype(jnp.bfloat16)
    w2b = w2.astype(jnp.bfloat16)
    ys = _grouped_mlp(blkexp[:G], xs, w1b, b1.reshape(E, 1, F),
                      w2b, b2.reshape(E, 1, D))

    if USE_SC:
        out2d = _run_combine(ys, dest, pw)
    else:
        out2d = _combine_jnp(ys, dest, pw)

    out = out2d.reshape(B, S, D)
    probs = pval[:2].T.reshape(B, S, K)
    return out, probs


# speedup vs baseline: 4.4724x; 1.0703x over previous
import jax
import jax.numpy as jnp
from jax import lax
from jax.experimental import pallas as pl
from jax.experimental.pallas import tpu as pltpu
from jax.experimental.pallas import tpu_sc as plsc

INTERPRET = False
USE_SC = True

E = 8
K = 2
D = 1024
F = 2048
T = 4096
EP = 128
BS = 256
G = T * K // BS + (E - 1)
P = G * BS
GP = 48
NW = 32
APW = T * K // NW
TPW = T // NW

NEG = -1e30


def _gelu(h):
    return 0.5 * h * (1.0 + jax.lax.erf(h * 0.7071067811865476))


def _router_body(x_ref, rw_ref, rb_ref, eidx_ref, pval_ref):
    xb = x_ref[...]
    rw = rw_ref[...]
    lg = lax.dot_general(
        rw.astype(jnp.bfloat16), xb.astype(jnp.bfloat16),
        (((0,), (1,)), ((), ())),
        preferred_element_type=jnp.float32)
    lg = lg + rb_ref[...]
    row = jax.lax.broadcasted_iota(jnp.int32, lg.shape, 0)
    lg = jnp.where(row < E, lg, NEG)
    v0 = jnp.max(lg, axis=0, keepdims=True)
    i0 = jnp.min(jnp.where(lg == v0, row, EP), axis=0, keepdims=True)
    lg1 = jnp.where(row == i0, NEG, lg)
    v1 = jnp.max(lg1, axis=0, keepdims=True)
    i1 = jnp.min(jnp.where(lg1 == v1, row, EP), axis=0, keepdims=True)
    p0 = 1.0 / (1.0 + jnp.exp(v1 - v0))
    p1 = 1.0 - p0
    zi = jnp.zeros_like(i0)
    zp = jnp.zeros_like(p0)
    eidx_ref[...] = jnp.concatenate([i0, i1, zi, zi, zi, zi, zi, zi], axis=0)
    pval_ref[...] = jnp.concatenate([p0, p1, zp, zp, zp, zp, zp, zp], axis=0)


def _router(x2d, rw_pad, rb_t):
    TB = 1024
    return pl.pallas_call(
        _router_body,
        grid=(T // TB,),
        in_specs=[
            pl.BlockSpec((TB, D), lambda i: (i, 0)),
            pl.BlockSpec((D, EP), lambda i: (0, 0)),
            pl.BlockSpec((EP, 1), lambda i: (0, 0)),
        ],
        out_specs=[
            pl.BlockSpec((8, TB), lambda i: (0, i)),
            pl.BlockSpec((8, TB), lambda i: (0, i)),
        ],
        out_shape=[
            jax.ShapeDtypeStruct((8, T), jnp.int32),
            jax.ShapeDtypeStruct((8, T), jnp.float32),
        ],
        interpret=INTERPRET,
    )(x2d, rw_pad, rb_t)


def _lane_iota():
    return lax.iota(jnp.int32, 16)


def _extract(vec, lane):
    return jnp.sum(jnp.where(_lane_iota() == lane, vec, 0))


def _extract_f(vec, lane):
    return jnp.sum(jnp.where(_lane_iota() == lane, vec, 0.0))


def _binning_kernel(easgn_hbm, x_hbm, xs_hbm, dest_hbm, blkexp_hbm,
                    easgn_v, dest_v, tok_a, tok_b, dst_a, dst_b,
                    rows_a, rows_b, blk_v, sem_ga, sem_gb, sem_sa, sem_sb):
    wid = lax.axis_index("s") * 2 + lax.axis_index("c")
    lane = _lane_iota()

    pltpu.sync_copy(easgn_hbm, easgn_v)

    NB = 32
    NBATCH = APW // NB
    gbase = wid * APW
    tok0 = jnp.bitwise_and(gbase, T - 1)
    dsts = [dst_a, dst_b]
    bufs = [rows_a, rows_b]
    gsems = [sem_ga, sem_gb]
    ssems = [sem_sa, sem_sb]

    pend_g = [None, None]
    pend_s = [None, None]

    def start_read(b):
        sl = b % 2
        pend_g[sl] = pltpu.async_copy(
            x_hbm.at[pl.ds(pl.multiple_of(tok0 + b * NB, NB), NB)],
            bufs[sl], gsems[sl])

    start_read(0)
    start_read(1)
    my_start = wid * (APW // 16)

    def scan_body(c, carry):
        acc, snap = carry
        snap = [jnp.where(c == my_start, a, s) for a, s in zip(acc, snap)]
        v = easgn_v[pl.ds(c * 16, 16)]
        acc = [a + jnp.where(v == e, 1, 0) for e, a in enumerate(acc)]
        return acc, snap

    zeros16 = jnp.zeros((16,), jnp.int32)
    acc, snap = lax.fori_loop(0, (T * K) // 16, scan_body,
                              ([zeros16] * E, [zeros16] * E))
    counts = jnp.zeros((16,), jnp.int32)
    prefix = jnp.zeros((16,), jnp.int32)
    for e in range(E):
        counts = counts + jnp.where(lane == e, jnp.sum(acc[e]), 0)
        prefix = prefix + jnp.where(lane == e, jnp.sum(snap[e]), 0)

    nblk = (counts + (BS - 1)) // BS
    cumblk = plsc.cumsum(nblk)
    excl = (cumblk - nblk) * BS
    base_vec = excl + prefix

    @pl.when(wid == 0)
    def _():
        cb = [_extract(cumblk, e) for e in range(E)]
        for cc in range(GP // 16):
            gvec = lane + 16 * cc
            val = jnp.zeros((16,), jnp.int32)
            for e in range(E):
                val = val + jnp.where(cb[e] <= gvec, 1, 0)
            blk_v[pl.ds(cc * 16, 16)] = jnp.minimum(val, E - 1)
        pltpu.sync_copy(blk_v, blkexp_hbm)

    base = [_extract(base_vec, e) for e in range(E)]
    running = [jnp.int32(0)] * E
    for c in range(APW // 16):
        v = easgn_v[pl.ds((my_start + c) * 16, 16)]
        destc = jnp.zeros((16,), jnp.int32)
        for e in range(E):
            m = v == e
            ones = jnp.where(m, 1, 0)
            r = plsc.cumsum(ones)
            pos = (base[e] + running[e] - 1) + r
            destc = jnp.where(m, pos, destc)
            running[e] = running[e] + jnp.max(r)
        dest_v[pl.ds(c * 16, 16)] = destc
    pltpu.sync_copy(dest_v, dest_hbm.at[pl.ds(wid * APW, APW)])


    def fill_dst(b, sl):
        for c in range(NB // 16):
            dsts[sl][pl.ds(c * 16, 16)] = dest_v[pl.ds(b * NB + c * 16, 16)]

    for b in range(NBATCH):
        sl = b % 2
        pend_g[sl].wait()
        fill_dst(b, sl)
        pend_s[sl] = pltpu.async_copy(bufs[sl], xs_hbm.at[dsts[sl]], ssems[sl])
        if b + 2 < NBATCH:
            pend_s[sl].wait()
            start_read(b + 2)
    for sl in (0, 1):
        if pend_s[sl] is not None:
            pend_s[sl].wait()
def _run_binning(easgn, x2d):
    mesh = plsc.VectorSubcoreMesh(core_axis_name="c", subcore_axis_name="s")
    kern = pl.kernel(
        _binning_kernel,
        mesh=mesh,
        compiler_params=pltpu.CompilerParams(needs_layout_passes=False),
        out_type=[
            jax.ShapeDtypeStruct((P, D), jnp.float32),
            jax.ShapeDtypeStruct((T * K,), jnp.int32),
            jax.ShapeDtypeStruct((GP,), jnp.int32),
        ],
        scratch_types=[
            pltpu.VMEM((T * K,), jnp.int32),
            pltpu.VMEM((APW,), jnp.int32),
            pltpu.VMEM((32,), jnp.int32),
            pltpu.VMEM((32,), jnp.int32),
            pltpu.VMEM((32,), jnp.int32),
            pltpu.VMEM((32,), jnp.int32),
            pltpu.VMEM((32, D), jnp.float32),
            pltpu.VMEM((32, D), jnp.float32),
            pltpu.VMEM((GP,), jnp.int32),
            pltpu.SemaphoreType.DMA,
            pltpu.SemaphoreType.DMA,
            pltpu.SemaphoreType.DMA,
            pltpu.SemaphoreType.DMA,
        ],
    )
    return kern(easgn, x2d)


def _binning_jnp(easgn, x2d):
    a = easgn
    onehot = (a[:, None] == jnp.arange(E)[None, :]).astype(jnp.int32)
    counts = jnp.sum(onehot, axis=0)
    nblk = (counts + BS - 1) // BS
    cumblk = jnp.cumsum(nblk)
    excl = (cumblk - nblk) * BS
    rank = jnp.cumsum(onehot, axis=0) - onehot
    dest = excl[a] + jnp.take_along_axis(rank, a[:, None], axis=1)[:, 0]
    xs = jnp.zeros((P, D), jnp.float32).at[dest].set(
        x2d[jnp.arange(T * K) % T])
    blkexp = jnp.minimum(
        jnp.sum(cumblk[None, :] <= jnp.arange(GP)[:, None], axis=1), E - 1
    ).astype(jnp.int32)
    return xs, dest, blkexp


def _mlp_body(be_ref, xs_ref, w1_ref, b1_ref, w2_ref, b2_ref, ys_ref):
    h = jnp.dot(xs_ref[...].astype(jnp.bfloat16), w1_ref[0],
                preferred_element_type=jnp.float32)
    h = _gelu(h + b1_ref[0])
    y = jnp.dot(h.astype(jnp.bfloat16), w2_ref[0],
                preferred_element_type=jnp.float32)
    ys_ref[...] = y + b2_ref[0]


def _grouped_mlp(blkexp, xs, w1b, b1, w2b, b2):
    grid_spec = pltpu.PrefetchScalarGridSpec(
        num_scalar_prefetch=1,
        grid=(G,),
        in_specs=[
            pl.BlockSpec((BS, D), lambda g, be: (g, 0)),
            pl.BlockSpec((1, D, F), lambda g, be: (be[g], 0, 0)),
            pl.BlockSpec((1, 1, F), lambda g, be: (be[g], 0, 0)),
            pl.BlockSpec((1, F, D), lambda g, be: (be[g], 0, 0)),
            pl.BlockSpec((1, 1, D), lambda g, be: (be[g], 0, 0)),
        ],
        out_specs=pl.BlockSpec((BS, D), lambda g, be: (g, 0)),
    )
    return pl.pallas_call(
        _mlp_body,
        grid_spec=grid_spec,
        out_shape=jax.ShapeDtypeStruct((P, D), jnp.float32),
        interpret=INTERPRET,
    )(blkexp, xs, w1b, b1, w2b, b2)


def _combine_kernel(ys_hbm, dest_hbm, pw_hbm, out_hbm,
                    d0_v, d1_v, p0_v, p1_v, r0a, r1a, r0b, r1b, oa, ob,
                    sg0a, sg1a, sg0b, sg1b, soa, sob):
    wid = lax.axis_index("s") * 2 + lax.axis_index("c")
    tb = wid * TPW
    pltpu.sync_copy(dest_hbm.at[pl.ds(tb, TPW)], d0_v)
    pltpu.sync_copy(dest_hbm.at[pl.ds(T + tb, TPW)], d1_v)
    pltpu.sync_copy(pw_hbm.at[pl.ds(tb, TPW)], p0_v)
    pltpu.sync_copy(pw_hbm.at[pl.ds(T + tb, TPW)], p1_v)
    r0s = [r0a, r0b]
    r1s = [r1a, r1b]
    outs = [oa, ob]
    g0s = [sg0a, sg0b]
    g1s = [sg1a, sg1b]
    osems = [soa, sob]
    NC = TPW // 16
    pend_g = [None, None]
    pend_o = [None, None]

    def start_gathers(c):
        sl = c % 2
        i0 = d0_v[pl.ds(c * 16, 16)]
        i1 = d1_v[pl.ds(c * 16, 16)]
        pend_g[sl] = (pltpu.async_copy(ys_hbm.at[i0], r0s[sl], g0s[sl]),
                      pltpu.async_copy(ys_hbm.at[i1], r1s[sl], g1s[sl]))

    start_gathers(0)
    start_gathers(1)
    for c in range(NC):
        sl = c % 2
        pend_g[sl][0].wait()
        pend_g[sl][1].wait()
        pa = p0_v[pl.ds(c * 16, 16)]
        pb = p1_v[pl.ds(c * 16, 16)]
        if pend_o[sl] is not None:
            pend_o[sl].wait()
        o_v = outs[sl]
        r0_v = r0s[sl]
        r1_v = r1s[sl]
        for i in range(16):
            s0 = _extract_f(pa, i)
            s1 = _extract_f(pb, i)

            def col_body(j, _):
                for u in range(4):
                    o_v[i, pl.ds(j * 64 + u * 16, 16)] = (
                        r0_v[i, pl.ds(j * 64 + u * 16, 16)] * s0
                        + r1_v[i, pl.ds(j * 64 + u * 16, 16)] * s1)
                return 0

            lax.fori_loop(0, D // 64, col_body, 0)
        if c + 2 < NC:
            start_gathers(c + 2)
        pend_o[sl] = pltpu.async_copy(
            o_v, out_hbm.at[pl.ds(tb + c * 16, 16)], osems[sl])
    for sl in (0, 1):
        if pend_o[sl] is not None:
            pend_o[sl].wait()


def _run_combine(ys, dest, pw):
    mesh = plsc.VectorSubcoreMesh(core_axis_name="c", subcore_axis_name="s")
    kern = pl.kernel(
        _combine_kernel,
        mesh=mesh,
        compiler_params=pltpu.CompilerParams(needs_layout_passes=False),
        out_type=jax.ShapeDtypeStruct((T, D), jnp.float32),
        scratch_types=[
            pltpu.VMEM((TPW,), jnp.int32),
            pltpu.VMEM((TPW,), jnp.int32),
            pltpu.VMEM((TPW,), jnp.float32),
            pltpu.VMEM((TPW,), jnp.float32),
            pltpu.VMEM((16, D), jnp.float32),
            pltpu.VMEM((16, D), jnp.float32),
            pltpu.VMEM((16, D), jnp.float32),
            pltpu.VMEM((16, D), jnp.float32),
            pltpu.VMEM((16, D), jnp.float32),
            pltpu.VMEM((16, D), jnp.float32),
            pltpu.SemaphoreType.DMA,
            pltpu.SemaphoreType.DMA,
            pltpu.SemaphoreType.DMA,
            pltpu.SemaphoreType.DMA,
            pltpu.SemaphoreType.DMA,
            pltpu.SemaphoreType.DMA,
        ],
    )
    return kern(ys, dest, pw)


def _combine_jnp(ys, dest, pw):
    r0 = ys[dest[:T]]
    r1 = ys[dest[T:]]
    return r0 * pw[:T, None] + r1 * pw[T:, None]


def kernel(x, router_w, router_b, w1, b1, w2, b2):
    B, S, _ = x.shape
    x2d = x.reshape(T, D)
    rw_pad = jnp.pad(router_w, ((0, 0), (0, EP - E)))
    rb_t = jnp.pad(router_b, (0, EP - E)).reshape(EP, 1)

    eidx, pval = _router(x2d, rw_pad, rb_t)

    easgn = eidx[:2].reshape(T * K)
    pw = pval[:2].reshape(T * K)

    if USE_SC:
        xs, dest, blkexp = _run_binning(easgn, x2d)
    else:
        xs, dest, blkexp = _binning_jnp(easgn, x2d)

    w1b = w1.astype(jnp.bfloat16)
    w2b = w2.astype(jnp.bfloat16)
    ys = _grouped_mlp(blkexp[:G], xs, w1b, b1.reshape(E, 1, F),
                      w2b, b2.reshape(E, 1, D))

    if USE_SC:
        out2d = _run_combine(ys, dest, pw)
    else:
        out2d = _combine_jnp(ys, dest, pw)

    out = out2d.reshape(B, S, D)
    probs = pval[:2].T.reshape(B, S, K)
    return out, probs


# speedup vs baseline: 5.0061x; 1.1193x over previous
import jax
import jax.numpy as jnp
from jax import lax
from jax.experimental import pallas as pl
from jax.experimental.pallas import tpu as pltpu
from jax.experimental.pallas import tpu_sc as plsc

INTERPRET = False
USE_SC = True

E = 8
K = 2
D = 1024
F = 2048
T = 4096
EP = 128
BS = 256
G = T * K // BS + (E - 1)
P = G * BS
GP = 48
NW = 32
APW = T * K // NW
TPW = T // NW

NEG = -1e30


def _gelu(h):
    return 0.5 * h * (1.0 + jax.lax.erf(h * 0.7071067811865476))


def _router_body(x_ref, rw_ref, rb_ref, eidx_ref, pval_ref):
    xb = x_ref[...]
    rw = rw_ref[...]
    lg = lax.dot_general(
        rw.astype(jnp.bfloat16), xb.astype(jnp.bfloat16),
        (((0,), (1,)), ((), ())),
        preferred_element_type=jnp.float32)
    lg = lg + rb_ref[...]
    row = jax.lax.broadcasted_iota(jnp.int32, lg.shape, 0)
    lg = jnp.where(row < E, lg, NEG)
    v0 = jnp.max(lg, axis=0, keepdims=True)
    i0 = jnp.min(jnp.where(lg == v0, row, EP), axis=0, keepdims=True)
    lg1 = jnp.where(row == i0, NEG, lg)
    v1 = jnp.max(lg1, axis=0, keepdims=True)
    i1 = jnp.min(jnp.where(lg1 == v1, row, EP), axis=0, keepdims=True)
    p0 = 1.0 / (1.0 + jnp.exp(v1 - v0))
    p1 = 1.0 - p0
    zi = jnp.zeros_like(i0)
    zp = jnp.zeros_like(p0)
    eidx_ref[...] = jnp.concatenate([i0, i1, zi, zi, zi, zi, zi, zi], axis=0)
    pval_ref[...] = jnp.concatenate([p0, p1, zp, zp, zp, zp, zp, zp], axis=0)


def _router(x2d, rw_pad, rb_t):
    TB = 1024
    return pl.pallas_call(
        _router_body,
        grid=(T // TB,),
        in_specs=[
            pl.BlockSpec((TB, D), lambda i: (i, 0)),
            pl.BlockSpec((D, EP), lambda i: (0, 0)),
            pl.BlockSpec((EP, 1), lambda i: (0, 0)),
        ],
        out_specs=[
            pl.BlockSpec((8, TB), lambda i: (0, i)),
            pl.BlockSpec((8, TB), lambda i: (0, i)),
        ],
        out_shape=[
            jax.ShapeDtypeStruct((8, T), jnp.int32),
            jax.ShapeDtypeStruct((8, T), jnp.float32),
        ],
        interpret=INTERPRET,
    )(x2d, rw_pad, rb_t)


def _lane_iota():
    return lax.iota(jnp.int32, 16)


def _extract(vec, lane):
    return jnp.sum(jnp.where(_lane_iota() == lane, vec, 0))


def _extract_f(vec, lane):
    return jnp.sum(jnp.where(_lane_iota() == lane, vec, 0.0))


def _binning_kernel(easgn_hbm, x_hbm, xs_hbm, dest_hbm, blkexp_hbm,
                    easgn_v, dest_v, tok_a, tok_b, dst_a, dst_b,
                    rows_a, rows_b, blk_v, sem_ga, sem_gb, sem_sa, sem_sb):
    wid = lax.axis_index("s") * 2 + lax.axis_index("c")
    lane = _lane_iota()

    pltpu.sync_copy(easgn_hbm, easgn_v)

    NB = 32
    NBATCH = APW // NB
    gbase = wid * APW
    tok0 = jnp.bitwise_and(gbase, T - 1)
    dsts = [dst_a, dst_b]
    bufs = [rows_a, rows_b]
    gsems = [sem_ga, sem_gb]
    ssems = [sem_sa, sem_sb]

    pend_g = [None, None]
    pend_s = [None, None]

    def start_read(b):
        sl = b % 2
        pend_g[sl] = pltpu.async_copy(
            x_hbm.at[pl.ds(pl.multiple_of(tok0 + b * NB, NB), NB)],
            bufs[sl], gsems[sl])

    start_read(0)
    start_read(1)
    my_start = wid * (APW // 16)

    def scan_body(c, carry):
        acc, snap = carry
        snap = [jnp.where(c == my_start, a, s) for a, s in zip(acc, snap)]
        v = easgn_v[pl.ds(c * 16, 16)]
        acc = [a + jnp.where(v == e, 1, 0) for e, a in enumerate(acc)]
        return acc, snap

    zeros16 = jnp.zeros((16,), jnp.int32)
    acc, snap = lax.fori_loop(0, (T * K) // 16, scan_body,
                              ([zeros16] * E, [zeros16] * E))
    counts = jnp.zeros((16,), jnp.int32)
    prefix = jnp.zeros((16,), jnp.int32)
    for e in range(E):
        counts = counts + jnp.where(lane == e, jnp.sum(acc[e]), 0)
        prefix = prefix + jnp.where(lane == e, jnp.sum(snap[e]), 0)

    nblk = (counts + (BS - 1)) // BS
    cumblk = plsc.cumsum(nblk)
    excl = (cumblk - nblk) * BS
    base_vec = excl + prefix

    @pl.when(wid == 0)
    def _():
        cb = [_extract(cumblk, e) for e in range(E)]
        for cc in range(GP // 16):
            gvec = lane + 16 * cc
            val = jnp.zeros((16,), jnp.int32)
            for e in range(E):
                val = val + jnp.where(cb[e] <= gvec, 1, 0)
            blk_v[pl.ds(cc * 16, 16)] = jnp.minimum(val, E - 1)
        pltpu.sync_copy(blk_v, blkexp_hbm)

    base = [_extract(base_vec, e) for e in range(E)]
    running = [jnp.int32(0)] * E
    for c in range(APW // 16):
        v = easgn_v[pl.ds((my_start + c) * 16, 16)]
        destc = jnp.zeros((16,), jnp.int32)
        for e in range(E):
            m = v == e
            ones = jnp.where(m, 1, 0)
            r = plsc.cumsum(ones)
            pos = (base[e] + running[e] - 1) + r
            destc = jnp.where(m, pos, destc)
            running[e] = running[e] + jnp.max(r)
        dest_v[pl.ds(c * 16, 16)] = destc
    pltpu.sync_copy(dest_v, dest_hbm.at[pl.ds(wid * APW, APW)])


    def fill_dst(b, sl):
        for c in range(NB // 16):
            dsts[sl][pl.ds(c * 16, 16)] = dest_v[pl.ds(b * NB + c * 16, 16)]

    for b in range(NBATCH):
        sl = b % 2
        pend_g[sl].wait()
        fill_dst(b, sl)
        pend_s[sl] = pltpu.async_copy(bufs[sl], xs_hbm.at[dsts[sl]], ssems[sl])
        if b + 2 < NBATCH:
            pend_s[sl].wait()
            start_read(b + 2)
    for sl in (0, 1):
        if pend_s[sl] is not None:
            pend_s[sl].wait()
def _run_binning(easgn, x2d):
    mesh = plsc.VectorSubcoreMesh(core_axis_name="c", subcore_axis_name="s")
    kern = pl.kernel(
        _binning_kernel,
        mesh=mesh,
        compiler_params=pltpu.CompilerParams(needs_layout_passes=False),
        out_type=[
            jax.ShapeDtypeStruct((P, D), jnp.float32),
            jax.ShapeDtypeStruct((T * K,), jnp.int32),
            jax.ShapeDtypeStruct((GP,), jnp.int32),
        ],
        scratch_types=[
            pltpu.VMEM((T * K,), jnp.int32),
            pltpu.VMEM((APW,), jnp.int32),
            pltpu.VMEM((32,), jnp.int32),
            pltpu.VMEM((32,), jnp.int32),
            pltpu.VMEM((32,), jnp.int32),
            pltpu.VMEM((32,), jnp.int32),
            pltpu.VMEM((32, D), jnp.float32),
            pltpu.VMEM((32, D), jnp.float32),
            pltpu.VMEM((GP,), jnp.int32),
            pltpu.SemaphoreType.DMA,
            pltpu.SemaphoreType.DMA,
            pltpu.SemaphoreType.DMA,
            pltpu.SemaphoreType.DMA,
        ],
    )
    return kern(easgn, x2d)


def _binning_jnp(easgn, x2d):
    a = easgn
    onehot = (a[:, None] == jnp.arange(E)[None, :]).astype(jnp.int32)
    counts = jnp.sum(onehot, axis=0)
    nblk = (counts + BS - 1) // BS
    cumblk = jnp.cumsum(nblk)
    excl = (cumblk - nblk) * BS
    rank = jnp.cumsum(onehot, axis=0) - onehot
    dest = excl[a] + jnp.take_along_axis(rank, a[:, None], axis=1)[:, 0]
    xs = jnp.zeros((P, D), jnp.float32).at[dest].set(
        x2d[jnp.arange(T * K) % T])
    blkexp = jnp.minimum(
        jnp.sum(cumblk[None, :] <= jnp.arange(GP)[:, None], axis=1), E - 1
    ).astype(jnp.int32)
    return xs, dest, blkexp


def _mlp_body(be_ref, xs_ref, w1_ref, b1_ref, w2_ref, b2_ref, ys_ref,
              w1c, w2c):
    g = pl.program_id(0)
    prev = be_ref[jnp.maximum(g - 1, 0)]
    changed = jnp.logical_or(g == 0, be_ref[g] != prev)

    @pl.when(changed)
    def _():
        w1c[...] = w1_ref[0].astype(jnp.bfloat16)
        w2c[...] = w2_ref[0].astype(jnp.bfloat16)

    h = jnp.dot(xs_ref[...].astype(jnp.bfloat16), w1c[...],
                preferred_element_type=jnp.float32)
    h = _gelu(h + b1_ref[0])
    y = jnp.dot(h.astype(jnp.bfloat16), w2c[...],
                preferred_element_type=jnp.float32)
    ys_ref[...] = y + b2_ref[0]


def _grouped_mlp(blkexp, xs, w1, b1, w2, b2):
    grid_spec = pltpu.PrefetchScalarGridSpec(
        num_scalar_prefetch=1,
        grid=(G,),
        in_specs=[
            pl.BlockSpec((BS, D), lambda g, be: (g, 0)),
            pl.BlockSpec((1, D, F), lambda g, be: (be[g], 0, 0)),
            pl.BlockSpec((1, 1, F), lambda g, be: (be[g], 0, 0)),
            pl.BlockSpec((1, F, D), lambda g, be: (be[g], 0, 0)),
            pl.BlockSpec((1, 1, D), lambda g, be: (be[g], 0, 0)),
        ],
        out_specs=pl.BlockSpec((BS, D), lambda g, be: (g, 0)),
        scratch_shapes=[
            pltpu.VMEM((D, F), jnp.bfloat16),
            pltpu.VMEM((F, D), jnp.bfloat16),
        ],
    )
    return pl.pallas_call(
        _mlp_body,
        grid_spec=grid_spec,
        out_shape=jax.ShapeDtypeStruct((P, D), jnp.float32),
        interpret=INTERPRET,
    )(blkexp, xs, w1, b1, w2, b2)


def _combine_kernel(ys_hbm, dest_hbm, pw_hbm, out_hbm,
                    d0_v, d1_v, p0_v, p1_v, r0a, r1a, r0b, r1b, oa, ob,
                    sg0a, sg1a, sg0b, sg1b, soa, sob):
    wid = lax.axis_index("s") * 2 + lax.axis_index("c")
    tb = wid * TPW
    pltpu.sync_copy(dest_hbm.at[pl.ds(tb, TPW)], d0_v)
    pltpu.sync_copy(dest_hbm.at[pl.ds(T + tb, TPW)], d1_v)
    pltpu.sync_copy(pw_hbm.at[pl.ds(tb, TPW)], p0_v)
    pltpu.sync_copy(pw_hbm.at[pl.ds(T + tb, TPW)], p1_v)
    r0s = [r0a, r0b]
    r1s = [r1a, r1b]
    outs = [oa, ob]
    g0s = [sg0a, sg0b]
    g1s = [sg1a, sg1b]
    osems = [soa, sob]
    NC = TPW // 16
    pend_g = [None, None]
    pend_o = [None, None]

    def start_gathers(c):
        sl = c % 2
        i0 = d0_v[pl.ds(c * 16, 16)]
        i1 = d1_v[pl.ds(c * 16, 16)]
        pend_g[sl] = (pltpu.async_copy(ys_hbm.at[i0], r0s[sl], g0s[sl]),
                      pltpu.async_copy(ys_hbm.at[i1], r1s[sl], g1s[sl]))

    start_gathers(0)
    start_gathers(1)
    for c in range(NC):
        sl = c % 2
        pend_g[sl][0].wait()
        pend_g[sl][1].wait()
        pa = p0_v[pl.ds(c * 16, 16)]
        pb = p1_v[pl.ds(c * 16, 16)]
        if pend_o[sl] is not None:
            pend_o[sl].wait()
        o_v = outs[sl]
        r0_v = r0s[sl]
        r1_v = r1s[sl]
        for i in range(16):
            s0 = _extract_f(pa, i)
            s1 = _extract_f(pb, i)

            def col_body(j, _):
                for u in range(4):
                    o_v[i, pl.ds(j * 64 + u * 16, 16)] = (
                        r0_v[i, pl.ds(j * 64 + u * 16, 16)] * s0
                        + r1_v[i, pl.ds(j * 64 + u * 16, 16)] * s1)
                return 0

            lax.fori_loop(0, D // 64, col_body, 0)
        if c + 2 < NC:
            start_gathers(c + 2)
        pend_o[sl] = pltpu.async_copy(
            o_v, out_hbm.at[pl.ds(tb + c * 16, 16)], osems[sl])
    for sl in (0, 1):
        if pend_o[sl] is not None:
            pend_o[sl].wait()


def _run_combine(ys, dest, pw):
    mesh = plsc.VectorSubcoreMesh(core_axis_name="c", subcore_axis_name="s")
    kern = pl.kernel(
        _combine_kernel,
        mesh=mesh,
        compiler_params=pltpu.CompilerParams(needs_layout_passes=False),
        out_type=jax.ShapeDtypeStruct((T, D), jnp.float32),
        scratch_types=[
            pltpu.VMEM((TPW,), jnp.int32),
            pltpu.VMEM((TPW,), jnp.int32),
            pltpu.VMEM((TPW,), jnp.float32),
            pltpu.VMEM((TPW,), jnp.float32),
            pltpu.VMEM((16, D), jnp.float32),
            pltpu.VMEM((16, D), jnp.float32),
            pltpu.VMEM((16, D), jnp.float32),
            pltpu.VMEM((16, D), jnp.float32),
            pltpu.VMEM((16, D), jnp.float32),
            pltpu.VMEM((16, D), jnp.float32),
            pltpu.SemaphoreType.DMA,
            pltpu.SemaphoreType.DMA,
            pltpu.SemaphoreType.DMA,
            pltpu.SemaphoreType.DMA,
            pltpu.SemaphoreType.DMA,
            pltpu.SemaphoreType.DMA,
        ],
    )
    return kern(ys, dest, pw)


def _combine_jnp(ys, dest, pw):
    r0 = ys[dest[:T]]
    r1 = ys[dest[T:]]
    return r0 * pw[:T, None] + r1 * pw[T:, None]


def kernel(x, router_w, router_b, w1, b1, w2, b2):
    B, S, _ = x.shape
    x2d = x.reshape(T, D)
    rw_pad = jnp.pad(router_w, ((0, 0), (0, EP - E)))
    rb_t = jnp.pad(router_b, (0, EP - E)).reshape(EP, 1)

    eidx, pval = _router(x2d, rw_pad, rb_t)

    easgn = eidx[:2].reshape(T * K)
    pw = pval[:2].reshape(T * K)

    if USE_SC:
        xs, dest, blkexp = _run_binning(easgn, x2d)
    else:
        xs, dest, blkexp = _binning_jnp(easgn, x2d)

    ys = _grouped_mlp(blkexp[:G], xs, w1, b1.reshape(E, 1, F),
                      w2, b2.reshape(E, 1, D))

    if USE_SC:
        out2d = _run_combine(ys, dest, pw)
    else:
        out2d = _combine_jnp(ys, dest, pw)

    out = out2d.reshape(B, S, D)
    probs = pval[:2].T.reshape(B, S, K)
    return out, probs


# speedup vs baseline: 5.1436x; 1.0275x over previous
import jax
import jax.numpy as jnp
from jax import lax
from jax.experimental import pallas as pl
from jax.experimental.pallas import tpu as pltpu
from jax.experimental.pallas import tpu_sc as plsc

INTERPRET = False
USE_SC = True

E = 8
K = 2
D = 1024
F = 2048
T = 4096
EP = 128
BS = 256
G = T * K // BS + (E - 1)
P = G * BS
GP = 48
NW = 32
APW = T * K // NW
TPW = T // NW

NEG = -1e30


def _gelu(h):
    return 0.5 * h * (1.0 + jax.lax.erf(h * 0.7071067811865476))


def _router_body(x_ref, rw_ref, rb_ref, eidx_ref, pval_ref):
    xb = x_ref[...]
    rw = rw_ref[...]
    lg = lax.dot_general(
        rw.astype(jnp.bfloat16), xb.astype(jnp.bfloat16),
        (((0,), (1,)), ((), ())),
        preferred_element_type=jnp.float32)
    lg = lg + rb_ref[...]
    row = jax.lax.broadcasted_iota(jnp.int32, lg.shape, 0)
    lg = jnp.where(row < E, lg, NEG)
    v0 = jnp.max(lg, axis=0, keepdims=True)
    i0 = jnp.min(jnp.where(lg == v0, row, EP), axis=0, keepdims=True)
    lg1 = jnp.where(row == i0, NEG, lg)
    v1 = jnp.max(lg1, axis=0, keepdims=True)
    i1 = jnp.min(jnp.where(lg1 == v1, row, EP), axis=0, keepdims=True)
    p0 = 1.0 / (1.0 + jnp.exp(v1 - v0))
    p1 = 1.0 - p0
    zi = jnp.zeros_like(i0)
    zp = jnp.zeros_like(p0)
    eidx_ref[...] = jnp.concatenate([i0, i1, zi, zi, zi, zi, zi, zi], axis=0)
    pval_ref[...] = jnp.concatenate([p0, p1, zp, zp, zp, zp, zp, zp], axis=0)


def _router(x2d, rw_pad, rb_t):
    TB = 1024
    return pl.pallas_call(
        _router_body,
        grid=(T // TB,),
        in_specs=[
            pl.BlockSpec((TB, D), lambda i: (i, 0)),
            pl.BlockSpec((D, EP), lambda i: (0, 0)),
            pl.BlockSpec((EP, 1), lambda i: (0, 0)),
        ],
        out_specs=[
            pl.BlockSpec((8, TB), lambda i: (0, i)),
            pl.BlockSpec((8, TB), lambda i: (0, i)),
        ],
        out_shape=[
            jax.ShapeDtypeStruct((8, T), jnp.int32),
            jax.ShapeDtypeStruct((8, T), jnp.float32),
        ],
        interpret=INTERPRET,
    )(x2d, rw_pad, rb_t)


def _lane_iota():
    return lax.iota(jnp.int32, 16)


def _extract(vec, lane):
    return jnp.sum(jnp.where(_lane_iota() == lane, vec, 0))


def _extract_f(vec, lane):
    return jnp.sum(jnp.where(_lane_iota() == lane, vec, 0.0))


def _binning_kernel(easgn_hbm, x_hbm, xs_hbm, dest_hbm, blkexp_hbm,
                    easgn_v, dest_v, tok_a, tok_b, dst_a, dst_b,
                    rows_a, rows_b, blk_v, sem_ga, sem_gb, sem_sa, sem_sb):
    wid = lax.axis_index("s") * 2 + lax.axis_index("c")
    lane = _lane_iota()

    pltpu.sync_copy(easgn_hbm, easgn_v)

    NB = 32
    NBATCH = APW // NB
    gbase = wid * APW
    tok0 = jnp.bitwise_and(gbase, T - 1)
    dsts = [dst_a, dst_b]
    bufs = [rows_a, rows_b]
    gsems = [sem_ga, sem_gb]
    ssems = [sem_sa, sem_sb]

    pend_g = [None, None]
    pend_s = [None, None]

    def start_read(b):
        sl = b % 2
        pend_g[sl] = pltpu.async_copy(
            x_hbm.at[pl.ds(pl.multiple_of(tok0 + b * NB, NB), NB)],
            bufs[sl], gsems[sl])

    start_read(0)
    start_read(1)
    my_start = wid * (APW // 16)

    def scan_body(c, carry):
        acc, snap = carry
        snap = [jnp.where(c == my_start, a, s) for a, s in zip(acc, snap)]
        v = easgn_v[pl.ds(c * 16, 16)]
        acc = [a + jnp.where(v == e, 1, 0) for e, a in enumerate(acc)]
        return acc, snap

    zeros16 = jnp.zeros((16,), jnp.int32)
    acc, snap = lax.fori_loop(0, (T * K) // 16, scan_body,
                              ([zeros16] * E, [zeros16] * E))
    counts = jnp.zeros((16,), jnp.int32)
    prefix = jnp.zeros((16,), jnp.int32)
    for e in range(E):
        counts = counts + jnp.where(lane == e, jnp.sum(acc[e]), 0)
        prefix = prefix + jnp.where(lane == e, jnp.sum(snap[e]), 0)

    nblk = (counts + (BS - 1)) // BS
    cumblk = plsc.cumsum(nblk)
    excl = (cumblk - nblk) * BS
    base_vec = excl + prefix

    @pl.when(wid == 0)
    def _():
        cb = [_extract(cumblk, e) for e in range(E)]
        for cc in range(GP // 16):
            gvec = lane + 16 * cc
            val = jnp.zeros((16,), jnp.int32)
            for e in range(E):
                val = val + jnp.where(cb[e] <= gvec, 1, 0)
            blk_v[pl.ds(cc * 16, 16)] = jnp.minimum(val, E - 1)
        pltpu.sync_copy(blk_v, blkexp_hbm)

    base = [_extract(base_vec, e) for e in range(E)]
    running = [jnp.int32(0)] * E
    for c in range(APW // 16):
        v = easgn_v[pl.ds((my_start + c) * 16, 16)]
        destc = jnp.zeros((16,), jnp.int32)
        for e in range(E):
            m = v == e
            ones = jnp.where(m, 1, 0)
            r = plsc.cumsum(ones)
            pos = (base[e] + running[e] - 1) + r
            destc = jnp.where(m, pos, destc)
            running[e] = running[e] + jnp.max(r)
        dest_v[pl.ds(c * 16, 16)] = destc
    pltpu.sync_copy(dest_v, dest_hbm.at[pl.ds(wid * APW, APW)])


    def fill_dst(b, sl):
        for c in range(NB // 16):
            dsts[sl][pl.ds(c * 16, 16)] = dest_v[pl.ds(b * NB + c * 16, 16)]

    for b in range(NBATCH):
        sl = b % 2
        pend_g[sl].wait()
        fill_dst(b, sl)
        pend_s[sl] = pltpu.async_copy(bufs[sl], xs_hbm.at[dsts[sl]], ssems[sl])
        if b + 2 < NBATCH:
            pend_s[sl].wait()
            start_read(b + 2)
    for sl in (0, 1):
        if pend_s[sl] is not None:
            pend_s[sl].wait()
def _run_binning(easgn, x2d):
    mesh = plsc.VectorSubcoreMesh(core_axis_name="c", subcore_axis_name="s")
    kern = pl.kernel(
        _binning_kernel,
        mesh=mesh,
        compiler_params=pltpu.CompilerParams(needs_layout_passes=False),
        out_type=[
            jax.ShapeDtypeStruct((P, D), jnp.float32),
            jax.ShapeDtypeStruct((T * K,), jnp.int32),
            jax.ShapeDtypeStruct((GP,), jnp.int32),
        ],
        scratch_types=[
            pltpu.VMEM((T * K,), jnp.int32),
            pltpu.VMEM((APW,), jnp.int32),
            pltpu.VMEM((32,), jnp.int32),
            pltpu.VMEM((32,), jnp.int32),
            pltpu.VMEM((32,), jnp.int32),
            pltpu.VMEM((32,), jnp.int32),
            pltpu.VMEM((32, D), jnp.float32),
            pltpu.VMEM((32, D), jnp.float32),
            pltpu.VMEM((GP,), jnp.int32),
            pltpu.SemaphoreType.DMA,
            pltpu.SemaphoreType.DMA,
            pltpu.SemaphoreType.DMA,
            pltpu.SemaphoreType.DMA,
        ],
    )
    return kern(easgn, x2d)


def _binning_jnp(easgn, x2d):
    a = easgn
    onehot = (a[:, None] == jnp.arange(E)[None, :]).astype(jnp.int32)
    counts = jnp.sum(onehot, axis=0)
    nblk = (counts + BS - 1) // BS
    cumblk = jnp.cumsum(nblk)
    excl = (cumblk - nblk) * BS
    rank = jnp.cumsum(onehot, axis=0) - onehot
    dest = excl[a] + jnp.take_along_axis(rank, a[:, None], axis=1)[:, 0]
    xs = jnp.zeros((P, D), jnp.float32).at[dest].set(
        x2d[jnp.arange(T * K) % T])
    blkexp = jnp.minimum(
        jnp.sum(cumblk[None, :] <= jnp.arange(GP)[:, None], axis=1), E - 1
    ).astype(jnp.int32)
    return xs, dest, blkexp


def _mlp_body(be_ref, xs_ref, w1_ref, b1_ref, w2_ref, b2_ref, ys_ref,
              w1c, w2c):
    g = pl.program_id(0)
    prev = be_ref[jnp.maximum(g - 1, 0)]
    changed = jnp.logical_or(g == 0, be_ref[g] != prev)

    @pl.when(changed)
    def _():
        w1c[...] = w1_ref[0].astype(jnp.bfloat16)
        w2c[...] = w2_ref[0].astype(jnp.bfloat16)

    F2 = F // 2
    xb = xs_ref[...].astype(jnp.bfloat16)
    h1 = jnp.dot(xb, w1c[:, :F2], preferred_element_type=jnp.float32)
    h2 = jnp.dot(xb, w1c[:, F2:], preferred_element_type=jnp.float32)
    g1 = _gelu(h1 + b1_ref[0, :, :F2]).astype(jnp.bfloat16)
    y1 = jnp.dot(g1, w2c[:F2], preferred_element_type=jnp.float32)
    g2 = _gelu(h2 + b1_ref[0, :, F2:]).astype(jnp.bfloat16)
    y2 = jnp.dot(g2, w2c[F2:], preferred_element_type=jnp.float32)
    y = (y1 + y2) + b2_ref[0]

    def _rne(v):
        b = lax.bitcast_convert_type(v, jnp.int32)
        return (b + 0x7FFF + ((b >> 16) & 1)) & jnp.int32(-65536)

    lo = _rne(y[:, :D // 2])
    hi = _rne(y[:, D // 2:])
    ys_ref[...] = jnp.bitwise_or(
        lax.shift_right_logical(lo, 16), hi)


def _grouped_mlp(blkexp, xs, w1, b1, w2, b2):
    grid_spec = pltpu.PrefetchScalarGridSpec(
        num_scalar_prefetch=1,
        grid=(G,),
        in_specs=[
            pl.BlockSpec((BS, D), lambda g, be: (g, 0)),
            pl.BlockSpec((1, D, F), lambda g, be: (be[g], 0, 0)),
            pl.BlockSpec((1, 1, F), lambda g, be: (be[g], 0, 0)),
            pl.BlockSpec((1, F, D), lambda g, be: (be[g], 0, 0)),
            pl.BlockSpec((1, 1, D), lambda g, be: (be[g], 0, 0)),
        ],
        out_specs=pl.BlockSpec((BS, D // 2), lambda g, be: (g, 0)),
        scratch_shapes=[
            pltpu.VMEM((D, F), jnp.bfloat16),
            pltpu.VMEM((F, D), jnp.bfloat16),
        ],
    )
    return pl.pallas_call(
        _mlp_body,
        grid_spec=grid_spec,
        out_shape=jax.ShapeDtypeStruct((P, D // 2), jnp.int32),
        interpret=INTERPRET,
    )(blkexp, xs, w1, b1, w2, b2)


def _combine_kernel(ys_hbm, dest_hbm, pw_hbm, out_hbm,
                    d0_v, d1_v, p0_v, p1_v, r0a, r1a, r0b, r1b, oa, ob,
                    sg0a, sg1a, sg0b, sg1b, soa, sob):
    wid = lax.axis_index("s") * 2 + lax.axis_index("c")
    tb = wid * TPW
    pltpu.sync_copy(dest_hbm.at[pl.ds(tb, TPW)], d0_v)
    pltpu.sync_copy(dest_hbm.at[pl.ds(T + tb, TPW)], d1_v)
    pltpu.sync_copy(pw_hbm.at[pl.ds(tb, TPW)], p0_v)
    pltpu.sync_copy(pw_hbm.at[pl.ds(T + tb, TPW)], p1_v)
    r0s = [r0a, r0b]
    r1s = [r1a, r1b]
    outs = [oa, ob]
    g0s = [sg0a, sg0b]
    g1s = [sg1a, sg1b]
    osems = [soa, sob]
    NC = TPW // 16
    lane = _lane_iota()
    himask = jnp.int32(-65536)
    pend_g = [None, None]
    pend_o = [None, None]

    def start_gathers(c):
        sl = c % 2
        i0 = d0_v[pl.ds(c * 16, 16)]
        i1 = d1_v[pl.ds(c * 16, 16)]
        pend_g[sl] = (pltpu.async_copy(ys_hbm.at[i0], r0s[sl], g0s[sl]),
                      pltpu.async_copy(ys_hbm.at[i1], r1s[sl], g1s[sl]))

    start_gathers(0)
    start_gathers(1)
    for c in range(NC):
        sl = c % 2
        pend_g[sl][0].wait()
        pend_g[sl][1].wait()
        pa = p0_v[pl.ds(c * 16, 16)]
        pb = p1_v[pl.ds(c * 16, 16)]
        if pend_o[sl] is not None:
            pend_o[sl].wait()
        o_v = outs[sl]
        r0_v = r0s[sl]
        r1_v = r1s[sl]
        for i in range(16):
            s0 = _extract_f(pa, i)
            s1 = _extract_f(pb, i)

            def col_body(j, _):
                for u in range(2):
                    cpos = j * 32 + u * 16
                    wa = r0_v[i, pl.ds(cpos, 16)]
                    wb = r1_v[i, pl.ds(cpos, 16)]
                    ae = plsc.bitcast(wa << 16, jnp.float32)
                    ao = plsc.bitcast(wa & himask, jnp.float32)
                    be_ = plsc.bitcast(wb << 16, jnp.float32)
                    bo = plsc.bitcast(wb & himask, jnp.float32)
                    o_v[i, pl.ds(cpos, 16)] = ae * s0 + be_ * s1
                    o_v[i, pl.ds(D // 2 + cpos, 16)] = ao * s0 + bo * s1
                return 0

            lax.fori_loop(0, (D // 2) // 32, col_body, 0)
        if c + 2 < NC:
            start_gathers(c + 2)
        pend_o[sl] = pltpu.async_copy(
            o_v, out_hbm.at[pl.ds(tb + c * 16, 16)], osems[sl])
    for sl in (0, 1):
        if pend_o[sl] is not None:
            pend_o[sl].wait()


def _run_combine(ys, dest, pw):
    mesh = plsc.VectorSubcoreMesh(core_axis_name="c", subcore_axis_name="s")
    kern = pl.kernel(
        _combine_kernel,
        mesh=mesh,
        compiler_params=pltpu.CompilerParams(needs_layout_passes=False),
        out_type=jax.ShapeDtypeStruct((T, D), jnp.float32),
        scratch_types=[
            pltpu.VMEM((TPW,), jnp.int32),
            pltpu.VMEM((TPW,), jnp.int32),
            pltpu.VMEM((TPW,), jnp.float32),
            pltpu.VMEM((TPW,), jnp.float32),
            pltpu.VMEM((16, D // 2), jnp.int32),
            pltpu.VMEM((16, D // 2), jnp.int32),
            pltpu.VMEM((16, D // 2), jnp.int32),
            pltpu.VMEM((16, D // 2), jnp.int32),
            pltpu.VMEM((16, D), jnp.float32),
            pltpu.VMEM((16, D), jnp.float32),
            pltpu.SemaphoreType.DMA,
            pltpu.SemaphoreType.DMA,
            pltpu.SemaphoreType.DMA,
            pltpu.SemaphoreType.DMA,
            pltpu.SemaphoreType.DMA,
            pltpu.SemaphoreType.DMA,
        ],
    )
    return kern(ys, dest, pw)


def _combine_jnp(ys, dest, pw):
    lo = lax.bitcast_convert_type(ys << 16, jnp.float32)
    hi = lax.bitcast_convert_type(ys & jnp.int32(-65536), jnp.float32)
    yf = jnp.concatenate([lo, hi], axis=1)
    r0 = yf[dest[:T]]
    r1 = yf[dest[T:]]
    return r0 * pw[:T, None] + r1 * pw[T:, None]


def kernel(x, router_w, router_b, w1, b1, w2, b2):
    B, S, _ = x.shape
    x2d = x.reshape(T, D)
    rw_pad = jnp.pad(router_w, ((0, 0), (0, EP - E)))
    rb_t = jnp.pad(router_b, (0, EP - E)).reshape(EP, 1)

    eidx, pval = _router(x2d, rw_pad, rb_t)

    easgn = eidx[:2].reshape(T * K)
    pw = pval[:2].reshape(T * K)

    if USE_SC:
        xs, dest, blkexp = _run_binning(easgn, x2d)
    else:
        xs, dest, blkexp = _binning_jnp(easgn, x2d)

    ys = _grouped_mlp(blkexp[:G], xs, w1, b1.reshape(E, 1, F),
                      w2, b2.reshape(E, 1, D))

    if USE_SC:
        out2d = _run_combine(ys, dest, pw)
    else:
        out2d = _combine_jnp(ys, dest, pw)

    out = out2d.reshape(B, S, D)
    probs = pval[:2].T.reshape(B, S, K)
    return out, probs
